# baseline (speedup 1.0000x reference)
import functools

import jax
import jax.numpy as jnp
from jax import lax
from jax.experimental import pallas as pl
from jax.experimental.pallas import tpu as pltpu
from jax.experimental.pallas import tpu_sc as plsc

B = 16384
D = 64
L = 16
NC = 2
NS = 16
NW = NC * NS
BPW = B // NW
NCHUNK = 4
CH = BPW // NCHUNK


def _mf_body(user_hbm, item_hbm, uw_hbm, iw_hbm, out_hbm,
             uidx, iidx, urows, irows, partials, outv, sem):
    wid = lax.axis_index("s") * NC + lax.axis_index("c")
    base = wid * BPW

    for j in range(NCHUNK):
        pltpu.sync_copy(user_hbm.at[pl.ds(base + j * CH, CH)], uidx.at[j])
        pltpu.sync_copy(item_hbm.at[pl.ds(base + j * CH, CH)], iidx.at[j])

    copies = []
    for j in range(NCHUNK):
        copies.append(pltpu.async_copy(
            uw_hbm.at[uidx.at[j]], urows.at[pl.ds(j * CH, CH)], sem))
        copies.append(pltpu.async_copy(
            iw_hbm.at[iidx.at[j]], irows.at[pl.ds(j * CH, CH)], sem))
    for cp in copies:
        cp.wait()

    def row(r, carry):
        acc = urows[r, pl.ds(0, L)] * irows[r, pl.ds(0, L)]
        for g in range(1, D // L):
            acc = acc + urows[r, pl.ds(g * L, L)] * irows[r, pl.ds(g * L, L)]
        partials[pl.ds(r * L, L)] = acc
        return carry

    lax.fori_loop(0, BPW, row, 0)

    lane = lax.iota(jnp.int32, L)

    def group(g, carry):
        idx = g * (L * L) + lane * L
        acc = plsc.load_gather(partials, [idx])
        for c in range(1, L):
            acc = acc + plsc.load_gather(partials, [idx + c])
        outv[pl.ds(g * L, L)] = acc
        return carry

    lax.fori_loop(0, BPW // L, group, 0)

    pltpu.sync_copy(outv, out_hbm.at[pl.ds(base, BPW)])


@jax.jit
def _mf(user, item, user_emb_w, item_emb_w):
    mesh = plsc.VectorSubcoreMesh(
        core_axis_name="c", subcore_axis_name="s",
        num_cores=NC, num_subcores=NS)
    return pl.kernel(
        _mf_body,
        out_type=jax.ShapeDtypeStruct((B,), jnp.float32),
        mesh=mesh,
        compiler_params=pltpu.CompilerParams(
            needs_layout_passes=False, use_tc_tiling_on_sc=False),
        scratch_types=[
            pltpu.VMEM((NCHUNK, CH), jnp.int32),
            pltpu.VMEM((NCHUNK, CH), jnp.int32),
            pltpu.VMEM((BPW, D), jnp.float32),
            pltpu.VMEM((BPW, D), jnp.float32),
            pltpu.VMEM((BPW * L,), jnp.float32),
            pltpu.VMEM((BPW,), jnp.float32),
            pltpu.SemaphoreType.DMA,
        ],
    )(user, item, user_emb_w, item_emb_w)


def kernel(user, item, user_emb_w, item_emb_w):
    return _mf(user.astype(jnp.int32), item.astype(jnp.int32),
               user_emb_w, item_emb_w)


# speedup vs baseline: 1.2043x; 1.2043x over previous
import functools

import jax
import jax.numpy as jnp
from jax import lax
from jax.experimental import pallas as pl
from jax.experimental.pallas import tpu as pltpu
from jax.experimental.pallas import tpu_sc as plsc

B = 16384
D = 64
W = 2 * D
L = 16
NC = 2
NS = 16
NW = NC * NS
BPW = B // NW
NCHUNK = 4
CH = BPW // NCHUNK


def _mf_body(user_hbm, item_hbm, comb_hbm, out_hbm,
             uidx, iidx, urows, irows, partials, outv, sem):
    wid = lax.axis_index("s") * NC + lax.axis_index("c")
    base = wid * BPW

    for j in range(NCHUNK):
        pltpu.sync_copy(user_hbm.at[pl.ds(base + j * CH, CH)], uidx.at[j])
        pltpu.sync_copy(item_hbm.at[pl.ds(base + j * CH, CH)], iidx.at[j])

    lane = lax.iota(jnp.int32, L)

    for j in range(NCHUNK):
        cu = pltpu.async_copy(comb_hbm.at[uidx.at[j]], urows, sem)
        ci = pltpu.async_copy(comb_hbm.at[iidx.at[j]], irows, sem)
        cu.wait()
        ci.wait()

        def row(r, carry):
            acc = urows[r, pl.ds(0, L)] * irows[r, pl.ds(D, L)]
            for g in range(1, D // L):
                acc = acc + (urows[r, pl.ds(g * L, L)] *
                             irows[r, pl.ds(D + g * L, L)])
            partials[pl.ds(r * L, L)] = acc
            return carry

        lax.fori_loop(0, CH, row, 0)

        def group(g, carry):
            idx = g * (L * L) + lane * L
            acc = plsc.load_gather(partials, [idx])
            for c in range(1, L):
                acc = acc + plsc.load_gather(partials, [idx + c])
            outv[pl.ds(j * CH + g * L, L)] = acc
            return carry

        lax.fori_loop(0, CH // L, group, 0)

    pltpu.sync_copy(outv, out_hbm.at[pl.ds(base, BPW)])


@jax.jit
def _mf(user, item, comb):
    mesh = plsc.VectorSubcoreMesh(
        core_axis_name="c", subcore_axis_name="s",
        num_cores=NC, num_subcores=NS)
    return pl.kernel(
        _mf_body,
        out_type=jax.ShapeDtypeStruct((B,), jnp.float32),
        mesh=mesh,
        compiler_params=pltpu.CompilerParams(needs_layout_passes=False),
        scratch_types=[
            pltpu.VMEM((NCHUNK, CH), jnp.int32),
            pltpu.VMEM((NCHUNK, CH), jnp.int32),
            pltpu.VMEM((CH, W), jnp.float32),
            pltpu.VMEM((CH, W), jnp.float32),
            pltpu.VMEM((CH * L,), jnp.float32),
            pltpu.VMEM((BPW,), jnp.float32),
            pltpu.SemaphoreType.DMA,
        ],
    )(user, item, comb)


def kernel(user, item, user_emb_w, item_emb_w):
    comb = jnp.concatenate([user_emb_w, item_emb_w], axis=1)
    return _mf(user.astype(jnp.int32), item.astype(jnp.int32), comb)


# speedup vs baseline: 1.6040x; 1.3319x over previous
import functools

import jax
import jax.numpy as jnp
from jax import lax
from jax.experimental import pallas as pl
from jax.experimental.pallas import tpu as pltpu
from jax.experimental.pallas import tpu_sc as plsc

B = 16384
D = 64
NU = 1000000
L = 16
NC = 2
NS = 16
NW = NC * NS

R = 31232
CW = 256
NCHW = R // CW
LO31 = (NW - 1) * R
NCH31 = (NU - LO31) // CW
TAIL_LO = LO31 + NCH31 * CW
TAIL_W = NU - TAIL_LO
DUMP = B

BPW = B // NW
CH = 128


def _extract_body(idx_hbm, tab_hbm, tail_hbm, vecs_hbm,
                  idxv, wlp, wlu, clp, clu, chunk, tailbuf, staging, sem):
    wid = lax.axis_index("s") * NC + lax.axis_index("c")
    lo = wid * R
    hi = jnp.where(wid == NW - 1, NU, lo + R)

    pltpu.sync_copy(idx_hbm, idxv)
    lane = lax.iota(jnp.int32, L)

    def scanv(v, off):
        u = idxv[pl.ds(v * L, L)]
        m = (u >= lo) & (u < hi)
        plsc.store_compressed(wlp.at[pl.ds(off, L)], v * L + lane, mask=m)
        plsc.store_compressed(wlu.at[pl.ds(off, L)], u, mask=m)
        return off + plsc.all_reduce_population_count(m)[0]

    n_w = lax.fori_loop(0, B // L, scanv, 0)

    def do_chunk(buf, clo, cw):
        def cscan(g, off):
            valid = (g * L + lane) < n_w
            pv = wlp[pl.ds(g * L, L)]
            uv = wlu[pl.ds(g * L, L)]
            m = valid & (uv >= clo) & (uv < clo + cw)
            plsc.store_compressed(clp.at[pl.ds(off, L)], pv, mask=m)
            plsc.store_compressed(clu.at[pl.ds(off, L)], uv, mask=m)
            return off + plsc.all_reduce_population_count(m)[0]

        n_c = lax.fori_loop(0, (n_w + L - 1) // L, cscan, 0)

        def egroup(g, carry):
            valid = (g * L + lane) < n_c
            pv = clp[pl.ds(g * L, L)]
            uv = clu[pl.ds(g * L, L)]
            uloc = jnp.where(valid, uv - clo, 0)
            posd = jnp.where(valid, pv, DUMP + wid)
            for d in range(D):
                dsplat = jnp.full((L,), d, jnp.int32)
                val = plsc.load_gather(buf, [dsplat, uloc])
                plsc.store_scatter(staging, [lane, dsplat], val)
            pltpu.async_copy(staging, vecs_hbm.at[posd], sem).wait()
            return carry

        lax.fori_loop(0, (n_c + L - 1) // L, egroup, 0)

    nch = jnp.where(wid == NW - 1, NCH31, NCHW)

    def chunkloop(c, carry):
        clo = lo + c * CW
        pltpu.sync_copy(tab_hbm.at[:, pl.ds(clo, CW)], chunk)
        do_chunk(chunk, clo, CW)
        return carry

    lax.fori_loop(0, nch, chunkloop, 0)

    @pl.when(wid == NW - 1)
    def _tail():
        pltpu.sync_copy(tail_hbm, tailbuf)
        do_chunk(tailbuf, TAIL_LO, TAIL_W)


@jax.jit
def _extract(idx, tab_t, tail_t):
    mesh = plsc.VectorSubcoreMesh(
        core_axis_name="c", subcore_axis_name="s",
        num_cores=NC, num_subcores=NS)
    return pl.kernel(
        _extract_body,
        out_type=jax.ShapeDtypeStruct((B + NW, 2 * D), jnp.float32),
        mesh=mesh,
        compiler_params=pltpu.CompilerParams(needs_layout_passes=False),
        scratch_types=[
            pltpu.VMEM((B,), jnp.int32),
            pltpu.VMEM((B + L,), jnp.int32),
            pltpu.VMEM((B + L,), jnp.int32),
            pltpu.VMEM((B + L,), jnp.int32),
            pltpu.VMEM((B + L,), jnp.int32),
            pltpu.VMEM((D, CW), jnp.float32),
            pltpu.VMEM((D, TAIL_W), jnp.float32),
            pltpu.VMEM((L, 2 * D), jnp.float32),
            pltpu.SemaphoreType.DMA,
        ],
    )(idx, tab_t, tail_t)


def _dot_body(uvecs_hbm, ivecs_hbm, out_hbm,
              ub, ib, partials, outv, sem):
    wid = lax.axis_index("s") * NC + lax.axis_index("c")
    base = wid * BPW
    lane = lax.iota(jnp.int32, L)

    for j in range(BPW // CH):
        cu = pltpu.async_copy(
            uvecs_hbm.at[pl.ds(base + j * CH, CH), :], ub, sem)
        ci = pltpu.async_copy(
            ivecs_hbm.at[pl.ds(base + j * CH, CH), :], ib, sem)
        cu.wait()
        ci.wait()

        def row(r, carry):
            acc = ub[r, pl.ds(0, L)] * ib[r, pl.ds(0, L)]
            for g in range(1, D // L):
                acc = acc + ub[r, pl.ds(g * L, L)] * ib[r, pl.ds(g * L, L)]
            partials[pl.ds(r * L, L)] = acc
            return carry

        lax.fori_loop(0, CH, row, 0)

        def group(g, carry):
            idx = g * (L * L) + lane * L
            acc = plsc.load_gather(partials, [idx])
            for c in range(1, L):
                acc = acc + plsc.load_gather(partials, [idx + c])
            outv[pl.ds(j * CH + g * L, L)] = acc
            return carry

        lax.fori_loop(0, CH // L, group, 0)

    pltpu.sync_copy(outv, out_hbm.at[pl.ds(base, BPW)])


@jax.jit
def _dot(uvecs, ivecs):
    mesh = plsc.VectorSubcoreMesh(
        core_axis_name="c", subcore_axis_name="s",
        num_cores=NC, num_subcores=NS)
    return pl.kernel(
        _dot_body,
        out_type=jax.ShapeDtypeStruct((B,), jnp.float32),
        mesh=mesh,
        compiler_params=pltpu.CompilerParams(needs_layout_passes=False),
        scratch_types=[
            pltpu.VMEM((CH, 2 * D), jnp.float32),
            pltpu.VMEM((CH, 2 * D), jnp.float32),
            pltpu.VMEM((CH * L,), jnp.float32),
            pltpu.VMEM((BPW,), jnp.float32),
            pltpu.SemaphoreType.DMA,
        ],
    )(uvecs, ivecs)


def kernel(user, item, user_emb_w, item_emb_w):
    ut = user_emb_w.T
    it = item_emb_w.T
    uvecs = _extract(user.astype(jnp.int32), ut, ut[:, TAIL_LO:])
    ivecs = _extract(item.astype(jnp.int32), it, it[:, TAIL_LO:])
    return _dot(uvecs, ivecs)


# speedup vs baseline: 2.0399x; 1.2718x over previous
import functools

import jax
import jax.numpy as jnp
from jax import lax
from jax.experimental import pallas as pl
from jax.experimental.pallas import tpu as pltpu
from jax.experimental.pallas import tpu_sc as plsc

B = 16384
D = 64
NU = 1000000
L = 16
NC = 2
NS = 16
NW = NC * NS

R = 31232
CW = 256
NCHW = R // CW
LO31 = (NW - 1) * R
NCH31 = (NU - LO31) // CW
TAIL_LO = LO31 + NCH31 * CW
TAIL_W = NU - TAIL_LO
DUMP = B
NR = 4

BPW = B // NW
CH = 128


def _extract_body(idx_hbm, tab_hbm, tail_hbm, vecs_hbm,
                  idxv, wlp, wlu, cl, chunk2, tailbuf, staging,
                  sem_c, sem_s):
    wid = lax.axis_index("s") * NC + lax.axis_index("c")
    lo = wid * R
    hi = jnp.where(wid == NW - 1, NU, lo + R)

    pltpu.sync_copy(idx_hbm, idxv)
    lane = lax.iota(jnp.int32, L)

    def scanv(v, off):
        u = idxv[pl.ds(v * L, L)]
        m = (u >= lo) & (u < hi)
        plsc.store_compressed(wlp.at[pl.ds(off, L)], v * L + lane, mask=m)
        plsc.store_compressed(wlu.at[pl.ds(off, L)], u, mask=m)
        return off + plsc.all_reduce_population_count(m)[0]

    n_w = lax.fori_loop(0, B // L, scanv, 0)

    def do_chunk(buf, clo, cw, ro):
        def cscan(g, off):
            valid = (g * L + lane) < n_w
            pv = wlp[pl.ds(g * L, L)]
            uv = wlu[pl.ds(g * L, L)]
            m = valid & (uv >= clo) & (uv < clo + cw)
            packed = (uv - clo) | (pv << 8)
            plsc.store_compressed(cl.at[pl.ds(off, L)], packed, mask=m)
            return off + plsc.all_reduce_population_count(m)[0]

        n_c = lax.fori_loop(0, (n_w + L - 1) // L, cscan, 0)

        def egroup(g, ro):
            r, o = ro

            @pl.when(o >= NR)
            def _drain():
                pltpu.make_async_copy(
                    vecs_hbm.at[pl.ds(0, L), :], staging.at[0], sem_s).wait()

            o = jnp.where(o >= NR, o - 1, o)
            slot = lax.rem(r, NR)
            valid = (g * L + lane) < n_c
            packed = cl[pl.ds(g * L, L)]
            uloc = jnp.where(valid, packed & 255, 0)
            posd = jnp.where(valid, lax.shift_right_logical(packed, 8),
                             DUMP + wid)
            for d in range(D):
                dsplat = jnp.full((L,), d, jnp.int32)
                val = plsc.load_gather(buf, [dsplat, uloc])
                plsc.store_scatter(staging.at[slot], [lane, dsplat], val)
            pltpu.async_copy(staging.at[slot], vecs_hbm.at[posd], sem_s)
            return (r + 1, o + 1)

        return lax.fori_loop(0, (n_c + L - 1) // L, egroup, ro)

    nch = jnp.where(wid == NW - 1, NCH31, NCHW)

    pltpu.async_copy(tab_hbm.at[:, pl.ds(lo, CW)], chunk2.at[0], sem_c)

    def chunkloop(c, ro):
        pltpu.make_async_copy(
            tab_hbm.at[:, pl.ds(0, CW)], chunk2.at[0], sem_c).wait()

        @pl.when(c + 1 < nch)
        def _prefetch():
            pltpu.async_copy(
                tab_hbm.at[:, pl.ds(lo + (c + 1) * CW, CW)],
                chunk2.at[lax.rem(c + 1, 2)], sem_c)

        return do_chunk(chunk2.at[lax.rem(c, 2)], lo + c * CW, CW, ro)

    ro = lax.fori_loop(0, nch, chunkloop, (0, 0))

    @pl.when(wid == NW - 1)
    def _tail():
        pltpu.sync_copy(tail_hbm, tailbuf)
        r, o = do_chunk(tailbuf, TAIL_LO, TAIL_W, ro)
        _ = lax.fori_loop(0, o, lambda i, c: _drain_one(vecs_hbm, staging,
                                                        sem_s, c), 0)

    @pl.when(wid != NW - 1)
    def _nodrain():
        _, o = ro
        _ = lax.fori_loop(0, o, lambda i, c: _drain_one(vecs_hbm, staging,
                                                        sem_s, c), 0)


def _drain_one(vecs_hbm, staging, sem_s, c):
    pltpu.make_async_copy(
        vecs_hbm.at[pl.ds(0, L), :], staging.at[0], sem_s).wait()
    return c


@jax.jit
def _extract(idx, tab_t, tail_t):
    mesh = plsc.VectorSubcoreMesh(
        core_axis_name="c", subcore_axis_name="s",
        num_cores=NC, num_subcores=NS)
    return pl.kernel(
        _extract_body,
        out_type=jax.ShapeDtypeStruct((B + NW, 2 * D), jnp.float32),
        mesh=mesh,
        compiler_params=pltpu.CompilerParams(needs_layout_passes=False),
        scratch_types=[
            pltpu.VMEM((B,), jnp.int32),
            pltpu.VMEM((B + L,), jnp.int32),
            pltpu.VMEM((B + L,), jnp.int32),
            pltpu.VMEM((B + L,), jnp.int32),
            pltpu.VMEM((2, D, CW), jnp.float32),
            pltpu.VMEM((D, TAIL_W), jnp.float32),
            pltpu.VMEM((NR, L, 2 * D), jnp.float32),
            pltpu.SemaphoreType.DMA,
            pltpu.SemaphoreType.DMA,
        ],
    )(idx, tab_t, tail_t)


def _dot_body(uvecs_hbm, ivecs_hbm, out_hbm,
              ub, ib, partials, outv, sem):
    wid = lax.axis_index("s") * NC + lax.axis_index("c")
    base = wid * BPW
    lane = lax.iota(jnp.int32, L)

    for j in range(BPW // CH):
        cu = pltpu.async_copy(
            uvecs_hbm.at[pl.ds(base + j * CH, CH), :], ub, sem)
        ci = pltpu.async_copy(
            ivecs_hbm.at[pl.ds(base + j * CH, CH), :], ib, sem)
        cu.wait()
        ci.wait()

        def row(r, carry):
            acc = ub[r, pl.ds(0, L)] * ib[r, pl.ds(0, L)]
            for g in range(1, D // L):
                acc = acc + ub[r, pl.ds(g * L, L)] * ib[r, pl.ds(g * L, L)]
            partials[pl.ds(r * L, L)] = acc
            return carry

        lax.fori_loop(0, CH, row, 0)

        def group(g, carry):
            idx = g * (L * L) + lane * L
            acc = plsc.load_gather(partials, [idx])
            for c in range(1, L):
                acc = acc + plsc.load_gather(partials, [idx + c])
            outv[pl.ds(j * CH + g * L, L)] = acc
            return carry

        lax.fori_loop(0, CH // L, group, 0)

    pltpu.sync_copy(outv, out_hbm.at[pl.ds(base, BPW)])


@jax.jit
def _dot(uvecs, ivecs):
    mesh = plsc.VectorSubcoreMesh(
        core_axis_name="c", subcore_axis_name="s",
        num_cores=NC, num_subcores=NS)
    return pl.kernel(
        _dot_body,
        out_type=jax.ShapeDtypeStruct((B,), jnp.float32),
        mesh=mesh,
        compiler_params=pltpu.CompilerParams(needs_layout_passes=False),
        scratch_types=[
            pltpu.VMEM((CH, 2 * D), jnp.float32),
            pltpu.VMEM((CH, 2 * D), jnp.float32),
            pltpu.VMEM((CH * L,), jnp.float32),
            pltpu.VMEM((BPW,), jnp.float32),
            pltpu.SemaphoreType.DMA,
        ],
    )(uvecs, ivecs)


def kernel(user, item, user_emb_w, item_emb_w):
    ut = user_emb_w.T
    it = item_emb_w.T
    uvecs = _extract(user.astype(jnp.int32), ut, ut[:, TAIL_LO:])
    ivecs = _extract(item.astype(jnp.int32), it, it[:, TAIL_LO:])
    return _dot(uvecs, ivecs)


# speedup vs baseline: 2.0612x; 1.0105x over previous
import functools

import jax
import jax.numpy as jnp
from jax import lax
from jax.experimental import pallas as pl
from jax.experimental.pallas import tpu as pltpu
from jax.experimental.pallas import tpu_sc as plsc

B = 16384
D = 64
NU = 1000000
L = 16
NC = 2
NS = 16
NW = NC * NS

R = 31232
CW = 256
NCHW = R // CW
LO31 = (NW - 1) * R
NCH31 = (NU - LO31) // CW
TAIL_LO = LO31 + NCH31 * CW
TAIL_W = NU - TAIL_LO
DUMP = B
NR = 2
NBUF = 4

BPW = B // NW
CH = 128


def _extract_body(idx_hbm, tab_hbm, tail_hbm, vecs_hbm,
                  idxv, wlp, cl, chunkb, tailbuf, staging,
                  sem_c, sem_s):
    wid = lax.axis_index("s") * NC + lax.axis_index("c")
    lo = wid * R
    hi = jnp.where(wid == NW - 1, NU, lo + R)

    pltpu.sync_copy(idx_hbm, idxv)
    lane = lax.iota(jnp.int32, L)

    def scanv(v, off):
        u = idxv[pl.ds(v * L, L)]
        m = (u >= lo) & (u < hi)
        plsc.store_compressed(wlp.at[pl.ds(off, L)], v * L + lane, mask=m)
        return off + plsc.all_reduce_population_count(m)[0]

    n_w = lax.fori_loop(0, B // L, scanv, 0)

    def do_chunk(buf, clo, cw, ro):
        def cscan(g, off):
            valid = (g * L + lane) < n_w
            pv = wlp[pl.ds(g * L, L)] & (B - 1)
            uv = plsc.load_gather(idxv, [pv])
            m = valid & (uv >= clo) & (uv < clo + cw)
            packed = (uv - clo) | (pv << 8)
            plsc.store_compressed(cl.at[pl.ds(off, L)], packed, mask=m)
            return off + plsc.all_reduce_population_count(m)[0]

        n_c = lax.fori_loop(0, (n_w + L - 1) // L, cscan, 0)

        def egroup(g, ro):
            r, o = ro

            @pl.when(o >= NR)
            def _drain():
                pltpu.make_async_copy(
                    vecs_hbm.at[pl.ds(0, L), :], staging.at[0], sem_s).wait()

            o = jnp.where(o >= NR, o - 1, o)
            slot = lax.rem(r, NR)
            valid = (g * L + lane) < n_c
            packed = cl[pl.ds(g * L, L)]
            uloc = jnp.where(valid, packed & 255, 0)
            posd = jnp.where(valid, lax.shift_right_logical(packed, 8),
                             DUMP + wid)
            for d in range(D):
                dsplat = jnp.full((L,), d, jnp.int32)
                val = plsc.load_gather(buf, [dsplat, uloc])
                plsc.store_scatter(staging.at[slot], [lane, dsplat], val)
            pltpu.async_copy(staging.at[slot], vecs_hbm.at[posd], sem_s)
            return (r + 1, o + 1)

        return lax.fori_loop(0, (n_c + L - 1) // L, egroup, ro)

    nch = jnp.where(wid == NW - 1, NCH31, NCHW)

    for p in range(NBUF - 1):
        pltpu.async_copy(tab_hbm.at[:, pl.ds(lo + p * CW, CW)],
                         chunkb.at[p], sem_c)

    def chunkloop(c, ro):
        pltpu.make_async_copy(
            tab_hbm.at[:, pl.ds(0, CW)], chunkb.at[0], sem_c).wait()

        @pl.when(c + (NBUF - 1) < nch)
        def _prefetch():
            pltpu.async_copy(
                tab_hbm.at[:, pl.ds(lo + (c + (NBUF - 1)) * CW, CW)],
                chunkb.at[lax.rem(c + (NBUF - 1), NBUF)], sem_c)

        return do_chunk(chunkb.at[lax.rem(c, NBUF)], lo + c * CW, CW, ro)

    ro = lax.fori_loop(0, nch, chunkloop, (0, 0))

    @pl.when(wid == NW - 1)
    def _tail():
        pltpu.sync_copy(tail_hbm, tailbuf)
        r, o = do_chunk(tailbuf, TAIL_LO, TAIL_W, ro)
        _ = lax.fori_loop(0, o, lambda i, c: _drain_one(vecs_hbm, staging,
                                                        sem_s, c), 0)

    @pl.when(wid != NW - 1)
    def _nodrain():
        _, o = ro
        _ = lax.fori_loop(0, o, lambda i, c: _drain_one(vecs_hbm, staging,
                                                        sem_s, c), 0)


def _drain_one(vecs_hbm, staging, sem_s, c):
    pltpu.make_async_copy(
        vecs_hbm.at[pl.ds(0, L), :], staging.at[0], sem_s).wait()
    return c


@jax.jit
def _extract(idx, tab_t, tail_t):
    mesh = plsc.VectorSubcoreMesh(
        core_axis_name="c", subcore_axis_name="s",
        num_cores=NC, num_subcores=NS)
    return pl.kernel(
        _extract_body,
        out_type=jax.ShapeDtypeStruct((B + NW, 2 * D), jnp.float32),
        mesh=mesh,
        compiler_params=pltpu.CompilerParams(needs_layout_passes=False),
        scratch_types=[
            pltpu.VMEM((B,), jnp.int32),
            pltpu.VMEM((B + L,), jnp.int32),
            pltpu.VMEM((B + L,), jnp.int32),
            pltpu.VMEM((NBUF, D, CW), jnp.float32),
            pltpu.VMEM((D, TAIL_W), jnp.float32),
            pltpu.VMEM((NR, L, 2 * D), jnp.float32),
            pltpu.SemaphoreType.DMA,
            pltpu.SemaphoreType.DMA,
        ],
    )(idx, tab_t, tail_t)


def _dot_body(uvecs_hbm, ivecs_hbm, out_hbm,
              ub, ib, partials, outv, sem):
    wid = lax.axis_index("s") * NC + lax.axis_index("c")
    base = wid * BPW
    lane = lax.iota(jnp.int32, L)

    for j in range(BPW // CH):
        cu = pltpu.async_copy(
            uvecs_hbm.at[pl.ds(base + j * CH, CH), :], ub, sem)
        ci = pltpu.async_copy(
            ivecs_hbm.at[pl.ds(base + j * CH, CH), :], ib, sem)
        cu.wait()
        ci.wait()

        def row(r, carry):
            acc = ub[r, pl.ds(0, L)] * ib[r, pl.ds(0, L)]
            for g in range(1, D // L):
                acc = acc + ub[r, pl.ds(g * L, L)] * ib[r, pl.ds(g * L, L)]
            partials[pl.ds(r * L, L)] = acc
            return carry

        lax.fori_loop(0, CH, row, 0)

        def group(g, carry):
            idx = g * (L * L) + lane * L
            acc = plsc.load_gather(partials, [idx])
            for c in range(1, L):
                acc = acc + plsc.load_gather(partials, [idx + c])
            outv[pl.ds(j * CH + g * L, L)] = acc
            return carry

        lax.fori_loop(0, CH // L, group, 0)

    pltpu.sync_copy(outv, out_hbm.at[pl.ds(base, BPW)])


@jax.jit
def _dot(uvecs, ivecs):
    mesh = plsc.VectorSubcoreMesh(
        core_axis_name="c", subcore_axis_name="s",
        num_cores=NC, num_subcores=NS)
    return pl.kernel(
        _dot_body,
        out_type=jax.ShapeDtypeStruct((B,), jnp.float32),
        mesh=mesh,
        compiler_params=pltpu.CompilerParams(needs_layout_passes=False),
        scratch_types=[
            pltpu.VMEM((CH, 2 * D), jnp.float32),
            pltpu.VMEM((CH, 2 * D), jnp.float32),
            pltpu.VMEM((CH * L,), jnp.float32),
            pltpu.VMEM((BPW,), jnp.float32),
            pltpu.SemaphoreType.DMA,
        ],
    )(uvecs, ivecs)


def kernel(user, item, user_emb_w, item_emb_w):
    ut = user_emb_w.T
    it = item_emb_w.T
    uvecs = _extract(user.astype(jnp.int32), ut, ut[:, TAIL_LO:])
    ivecs = _extract(item.astype(jnp.int32), it, it[:, TAIL_LO:])
    return _dot(uvecs, ivecs)


# speedup vs baseline: 3.0121x; 1.4613x over previous
import functools

import jax
import jax.numpy as jnp
from jax import lax
from jax.experimental import pallas as pl
from jax.experimental.pallas import tpu as pltpu
from jax.experimental.pallas import tpu_sc as plsc

B = 16384
D = 64
NU = 1000000
L = 16
NC = 2
NS = 16
NW = NC * NS

R = 31232
CW = 512
NCHW = R // CW
LO31 = (NW - 1) * R
NCH31 = (NU - LO31) // CW
TAIL_LO = LO31 + NCH31 * CW
TAIL_W = NU - TAIL_LO
DUMP = B
NR = 2
NBUF = 2

BPW = B // NW
CH = 128


def _extract_body(idx_hbm, tab_hbm, tail_hbm, vecs_hbm,
                  idxv, wlp, cl, chunkb, tailbuf, staging,
                  sem_c, sem_s):
    wid = lax.axis_index("s") * NC + lax.axis_index("c")
    lo = wid * R
    hi = jnp.where(wid == NW - 1, NU, lo + R)

    pltpu.sync_copy(idx_hbm, idxv)
    lane = lax.iota(jnp.int32, L)

    def scanv(v, off):
        u = idxv[pl.ds(v * L, L)]
        m = (u >= lo) & (u < hi)
        plsc.store_compressed(wlp.at[pl.ds(off, L)], v * L + lane, mask=m)
        return off + plsc.all_reduce_population_count(m)[0]

    n_w = lax.fori_loop(0, B // L, scanv, 0)

    def do_chunk(buf, clo, cw, ro):
        def cscan(g, off):
            valid = (g * L + lane) < n_w
            pv = wlp[pl.ds(g * L, L)] & (B - 1)
            uv = plsc.load_gather(idxv, [pv])
            m = valid & (uv >= clo) & (uv < clo + cw)
            packed = (uv - clo) | (pv << 16)
            plsc.store_compressed(cl.at[pl.ds(off, L)], packed, mask=m)
            return off + plsc.all_reduce_population_count(m)[0]

        n_c = lax.fori_loop(0, (n_w + L - 1) // L, cscan, 0)

        def egroup(g, ro):
            r, o = ro

            @pl.when(o >= NR)
            def _drain():
                pltpu.make_async_copy(
                    vecs_hbm.at[pl.ds(0, L), :], staging.at[0], sem_s).wait()

            o = jnp.where(o >= NR, o - 1, o)
            slot = lax.rem(r, NR)
            valid = (g * L + lane) < n_c
            packed = cl[pl.ds(g * L, L)]
            uloc = jnp.where(valid, packed & 0xFFFF, 0)
            posd = jnp.where(valid, lax.shift_right_logical(packed, 16),
                             DUMP + wid)
            for d in range(D):
                dsplat = jnp.full((L,), d, jnp.int32)
                val = plsc.load_gather(buf, [dsplat, uloc])
                plsc.store_scatter(staging.at[slot], [lane, dsplat], val)
            pltpu.async_copy(staging.at[slot], vecs_hbm.at[posd], sem_s)
            return (r + 1, o + 1)

        return lax.fori_loop(0, (n_c + L - 1) // L, egroup, ro)

    nch = jnp.where(wid == NW - 1, NCH31, NCHW)

    for p in range(NBUF - 1):
        pltpu.async_copy(tab_hbm.at[:, pl.ds(lo + p * CW, CW)],
                         chunkb.at[p], sem_c)

    def chunkloop(c, ro):
        pltpu.make_async_copy(
            tab_hbm.at[:, pl.ds(0, CW)], chunkb.at[0], sem_c).wait()

        @pl.when(c + (NBUF - 1) < nch)
        def _prefetch():
            pltpu.async_copy(
                tab_hbm.at[:, pl.ds(lo + (c + (NBUF - 1)) * CW, CW)],
                chunkb.at[lax.rem(c + (NBUF - 1), NBUF)], sem_c)

        return do_chunk(chunkb.at[lax.rem(c, NBUF)], lo + c * CW, CW, ro)

    ro = lax.fori_loop(0, nch, chunkloop, (0, 0))

    @pl.when(wid == NW - 1)
    def _tail():
        pltpu.sync_copy(tail_hbm, tailbuf)
        r, o = do_chunk(tailbuf, TAIL_LO, TAIL_W, ro)
        _ = lax.fori_loop(0, o, lambda i, c: _drain_one(vecs_hbm, staging,
                                                        sem_s, c), 0)

    @pl.when(wid != NW - 1)
    def _nodrain():
        _, o = ro
        _ = lax.fori_loop(0, o, lambda i, c: _drain_one(vecs_hbm, staging,
                                                        sem_s, c), 0)


def _drain_one(vecs_hbm, staging, sem_s, c):
    pltpu.make_async_copy(
        vecs_hbm.at[pl.ds(0, L), :], staging.at[0], sem_s).wait()
    return c


@jax.jit
def _extract(idx, tab_t, tail_t):
    mesh = plsc.VectorSubcoreMesh(
        core_axis_name="c", subcore_axis_name="s",
        num_cores=NC, num_subcores=NS)
    return pl.kernel(
        _extract_body,
        out_type=jax.ShapeDtypeStruct((B + NW, 2 * D), jnp.float32),
        mesh=mesh,
        compiler_params=pltpu.CompilerParams(needs_layout_passes=False),
        scratch_types=[
            pltpu.VMEM((B,), jnp.int32),
            pltpu.VMEM((B + L,), jnp.int32),
            pltpu.VMEM((B + L,), jnp.int32),
            pltpu.VMEM((NBUF, D, CW), jnp.float32),
            pltpu.VMEM((D, TAIL_W), jnp.float32),
            pltpu.VMEM((NR, L, 2 * D), jnp.float32),
            pltpu.SemaphoreType.DMA,
            pltpu.SemaphoreType.DMA,
        ],
    )(idx, tab_t, tail_t)


def _dot_body(uvecs_hbm, ivecs_hbm, out_hbm,
              ub, ib, partials, outv, sem):
    wid = lax.axis_index("s") * NC + lax.axis_index("c")
    base = wid * BPW
    lane = lax.iota(jnp.int32, L)

    for j in range(BPW // CH):
        cu = pltpu.async_copy(
            uvecs_hbm.at[pl.ds(base + j * CH, CH), :], ub, sem)
        ci = pltpu.async_copy(
            ivecs_hbm.at[pl.ds(base + j * CH, CH), :], ib, sem)
        cu.wait()
        ci.wait()

        def row(r, carry):
            acc = ub[r, pl.ds(0, L)] * ib[r, pl.ds(0, L)]
            for g in range(1, D // L):
                acc = acc + ub[r, pl.ds(g * L, L)] * ib[r, pl.ds(g * L, L)]
            partials[pl.ds(r * L, L)] = acc
            return carry

        lax.fori_loop(0, CH, row, 0)

        def group(g, carry):
            idx = g * (L * L) + lane * L
            acc = plsc.load_gather(partials, [idx])
            for c in range(1, L):
                acc = acc + plsc.load_gather(partials, [idx + c])
            outv[pl.ds(j * CH + g * L, L)] = acc
            return carry

        lax.fori_loop(0, CH // L, group, 0)

    pltpu.sync_copy(outv, out_hbm.at[pl.ds(base, BPW)])


@jax.jit
def _dot(uvecs, ivecs):
    mesh = plsc.VectorSubcoreMesh(
        core_axis_name="c", subcore_axis_name="s",
        num_cores=NC, num_subcores=NS)
    return pl.kernel(
        _dot_body,
        out_type=jax.ShapeDtypeStruct((B,), jnp.float32),
        mesh=mesh,
        compiler_params=pltpu.CompilerParams(needs_layout_passes=False),
        scratch_types=[
            pltpu.VMEM((CH, 2 * D), jnp.float32),
            pltpu.VMEM((CH, 2 * D), jnp.float32),
            pltpu.VMEM((CH * L,), jnp.float32),
            pltpu.VMEM((BPW,), jnp.float32),
            pltpu.SemaphoreType.DMA,
        ],
    )(uvecs, ivecs)


def kernel(user, item, user_emb_w, item_emb_w):
    ut = user_emb_w.T
    it = item_emb_w.T
    uvecs = _extract(user.astype(jnp.int32), ut, ut[:, TAIL_LO:])
    ivecs = _extract(item.astype(jnp.int32), it, it[:, TAIL_LO:])
    return _dot(uvecs, ivecs)
